# Initial kernel scaffold; baseline (speedup 1.0000x reference)
import jax
import jax.numpy as jnp
from jax.experimental import pallas as pl


def kernel(pooled_output, centroids, labels, delta, param_ab, w):
    raise NotImplementedError("write your pallas kernel here")


# speedup vs baseline: 9.8235x; 9.8235x over previous
import functools

import jax
import jax.numpy as jnp
from jax.experimental import pallas as pl
from jax.experimental.pallas import tpu as pltpu

_L = 150
_LP = 152
_D = 768
_H = 384
_BS = 512


def _softplus(x):
    return jnp.maximum(x, 0.0) + jnp.log1p(jnp.exp(-jnp.abs(x)))


def _body(w_ref, lab_ref, cent_ref, tab_ref, drow_ref, x_ref,
          loss_ref, dsp_ref, acc_ref):
    i = pl.program_id(0)
    nb = pl.num_programs(0)

    @pl.when(i == 0)
    def _init():
        acc_ref[0] = 0.0
        acc_ref[1] = 0.0
        dsp_ref[...] = _softplus(drow_ref[...])

    lab = lab_ref[pl.ds(i, 1), :]
    iota = jax.lax.broadcasted_iota(jnp.int32, (_LP, _BS), 0)
    oh_t = (iota == lab).astype(jnp.float32)

    c = jax.lax.dot_general(oh_t, cent_ref[...],
                            (((0,), (0,)), ((), ())),
                            preferred_element_type=jnp.float32)
    diff = x_ref[...] - c
    sq = diff * diff
    s1 = jnp.sum(sq[:, :_H], axis=1, keepdims=True)
    s2 = jnp.sum(sq[:, _H:], axis=1, keepdims=True)
    z1 = jnp.sqrt(s1)
    z2 = jnp.sqrt(s2)

    g = jax.lax.dot_general(oh_t, tab_ref[...],
                            (((0,), (0,)), ((), ())),
                            preferred_element_type=jnp.float32)
    k1 = _softplus(g[:, 0:1])
    k2 = _softplus(g[:, 1:2])
    d = _softplus(g[:, 2:3])

    euc = z1 * k1 + z2 * k2
    pos = jnp.maximum(euc - d, 0.0)
    neg = jnp.maximum(d - euc, 0.0)
    acc_ref[0] += jnp.sum(pos)
    acc_ref[1] += jnp.sum(neg)

    @pl.when(i == nb - 1)
    def _fin():
        batch = nb * _BS
        loss_ref[0, 0] = (w_ref[0, 0] * acc_ref[0] + acc_ref[1]) / batch


def kernel(pooled_output, centroids, labels, delta, param_ab, w=1.0):
    batch, d = pooled_output.shape
    nb = batch // _BS

    cent_pad = jnp.zeros((_LP, _D), jnp.float32).at[:_L].set(centroids)
    tab = jnp.zeros((_LP, 128), jnp.float32)
    tab = tab.at[:_L, 0].set(param_ab[:, 0])
    tab = tab.at[:_L, 1].set(param_ab[:, 1])
    tab = tab.at[:_L, 2].set(delta)
    drow = jnp.zeros((1, _LP), jnp.float32).at[0, :_L].set(delta)
    lab2d = labels.astype(jnp.int32).reshape(nb, _BS)
    w_arr = jnp.asarray(w, jnp.float32).reshape(1, 1)

    loss, dsp_row = pl.pallas_call(
        _body,
        grid=(nb,),
        in_specs=[
            pl.BlockSpec(memory_space=pltpu.SMEM),
            pl.BlockSpec((nb, _BS), lambda i: (0, 0)),
            pl.BlockSpec((_LP, _D), lambda i: (0, 0)),
            pl.BlockSpec((_LP, 128), lambda i: (0, 0)),
            pl.BlockSpec((1, _LP), lambda i: (0, 0)),
            pl.BlockSpec((_BS, _D), lambda i: (i, 0)),
        ],
        out_specs=[
            pl.BlockSpec(memory_space=pltpu.SMEM),
            pl.BlockSpec((1, _LP), lambda i: (0, 0)),
        ],
        out_shape=[
            jax.ShapeDtypeStruct((1, 1), jnp.float32),
            jax.ShapeDtypeStruct((1, _LP), jnp.float32),
        ],
        scratch_shapes=[pltpu.SMEM((2,), jnp.float32)],
        compiler_params=pltpu.CompilerParams(
            dimension_semantics=("arbitrary",),
        ),
    )(w_arr, lab2d, cent_pad, tab, drow, pooled_output)

    return loss[0, 0], dsp_row[0, :_L]
